# baseline (speedup 1.0000x reference)
import functools

import jax
import jax.numpy as jnp
from jax import lax
from jax.experimental import pallas as pl
from jax.experimental.pallas import tpu as pltpu

_TMP = 0.1
_FMP = 0.1
_SPLIT = 2
_NBUF = 16
_LAT = 12


def _body(tm_ref, fm_ref, x_ref, o_ref, buf_ref, insems, outsems):
    nb = x_ref.shape[0]

    def step(b, _):
        slot = lax.rem(b, _NBUF)

        @pl.when(b < nb)
        def _issue_in():
            @pl.when(b >= _NBUF)
            def _free_slot():
                pltpu.make_async_copy(
                    buf_ref.at[slot], o_ref.at[b - _NBUF], outsems.at[slot]
                ).wait()

            pltpu.async_copy(
                x_ref.at[b], buf_ref.at[slot], insems.at[slot], priority=0
            )

        d = b - _LAT

        @pl.when((d >= 0) & (d < nb))
        def _drain():
            dslot = lax.rem(d, _NBUF)
            pltpu.make_async_copy(
                x_ref.at[d], buf_ref.at[dslot], insems.at[dslot]
            ).wait()
            x = buf_ref[dslot]
            tm = tm_ref[d]
            fm = fm_ref[d]
            buf_ref[dslot] = x * tm * fm
            pltpu.async_copy(
                buf_ref.at[dslot], o_ref.at[d], outsems.at[dslot], priority=1
            )

        return ()

    lax.fori_loop(0, nb + _LAT, step, (), unroll=False)
    for s in range(_NBUF):
        d = nb - _NBUF + s
        pltpu.make_async_copy(buf_ref.at[s], o_ref.at[d], outsems.at[s]).wait()


def kernel(spec):
    B, C, Fd, T = spec.shape
    tlen = int(T * _TMP)
    flen = int(Fd * _FMP)
    t0 = jax.random.randint(
        jax.random.fold_in(jax.random.key(1), 0), (B,), 0, max(1, T - tlen + 1)
    )
    f0 = jax.random.randint(
        jax.random.fold_in(jax.random.key(1), 1), (B,), 0, max(1, Fd - flen + 1)
    )
    tidx = jnp.arange(T)[None, :]
    tm = jnp.where((tidx >= t0[:, None]) & (tidx < (t0 + tlen)[:, None]), 0.0, 1.0)
    fidx = jnp.arange(Fd)[None, :]
    fm = jnp.where((fidx >= f0[:, None]) & (fidx < (f0 + flen)[:, None]), 0.0, 1.0)

    Fr = Fd // _SPLIT
    nchunks = B * _SPLIT
    tm = jnp.repeat(tm, _SPLIT, axis=0).astype(spec.dtype).reshape(nchunks, 1, T)
    fm = fm.astype(spec.dtype).reshape(nchunks, Fr, 1)

    x = spec.reshape(nchunks, Fr, T)
    out = pl.pallas_call(
        _body,
        in_specs=[
            pl.BlockSpec(memory_space=pltpu.VMEM),
            pl.BlockSpec(memory_space=pltpu.VMEM),
            pl.BlockSpec(memory_space=pl.ANY),
        ],
        out_specs=pl.BlockSpec(memory_space=pl.ANY),
        out_shape=jax.ShapeDtypeStruct(x.shape, x.dtype),
        scratch_shapes=[
            pltpu.VMEM((_NBUF, Fr, T), spec.dtype),
            pltpu.SemaphoreType.DMA((_NBUF,)),
            pltpu.SemaphoreType.DMA((_NBUF,)),
        ],
    )(tm, fm, x)
    return out.reshape(B, C, Fd, T)
